# baseline (device time: 740589 ns/iter reference)
import jax
import jax.numpy as jnp
from jax import lax
from jax.experimental import pallas as pl
from jax.experimental.pallas import tpu as pltpu

N_DEV = 16


def _compute_partial(x, Wg, Wu, Wd):
    m, d = x.shape
    h_per = Wg.shape[1]
    n_out = Wd.shape[1]
    nk = 8
    kblk = h_per // nk

    def body(x_ref, wg_ref, wu_ref, wd_ref, out_ref):
        k = pl.program_id(0)
        gate = jnp.dot(x_ref[...], wg_ref[...], preferred_element_type=jnp.float32)
        up = jnp.dot(x_ref[...], wu_ref[...], preferred_element_type=jnp.float32)
        h = gate * (up * jax.nn.sigmoid(up))
        acc = jnp.dot(h, wd_ref[...], preferred_element_type=jnp.float32)

        @pl.when(k == 0)
        def _():
            out_ref[...] = acc

        @pl.when(k != 0)
        def _():
            out_ref[...] += acc

    return pl.pallas_call(
        body,
        grid=(nk,),
        in_specs=[
            pl.BlockSpec((m, d), lambda k: (0, 0)),
            pl.BlockSpec((d, kblk), lambda k: (0, k)),
            pl.BlockSpec((d, kblk), lambda k: (0, k)),
            pl.BlockSpec((kblk, n_out), lambda k: (k, 0)),
        ],
        out_specs=pl.BlockSpec((m, n_out), lambda k: (0, 0)),
        out_shape=jax.ShapeDtypeStruct((m, n_out), jnp.float32),
        compiler_params=pltpu.CompilerParams(
            dimension_semantics=("arbitrary",),
        ),
    )(x, Wg, Wu, Wd)


def _ring_allreduce(partial):
    m, n = partial.shape

    def body(p_ref, out_ref, comm_ref, send_sems, recv_sems):
        my = lax.axis_index("i")
        left = lax.rem(my + N_DEV - 1, N_DEV)
        right = lax.rem(my + 1, N_DEV)

        barrier_sem = pltpu.get_barrier_semaphore()
        for nbr in (left, right):
            pl.semaphore_signal(
                barrier_sem,
                inc=1,
                device_id=(nbr,),
                device_id_type=pl.DeviceIdType.MESH,
            )
        pl.semaphore_wait(barrier_sem, 2)

        out_ref[...] = p_ref[...]
        comm_ref[0] = p_ref[...]

        for h in range(N_DEV - 1):
            send_slot = h % 2
            recv_slot = (h + 1) % 2
            rdma = pltpu.make_async_remote_copy(
                src_ref=comm_ref.at[send_slot],
                dst_ref=comm_ref.at[recv_slot],
                send_sem=send_sems.at[send_slot],
                recv_sem=recv_sems.at[recv_slot],
                device_id=(right,),
                device_id_type=pl.DeviceIdType.MESH,
            )
            rdma.start()
            rdma.wait()
            out_ref[...] += comm_ref[recv_slot]

    return pl.pallas_call(
        body,
        out_shape=jax.ShapeDtypeStruct((m, n), jnp.float32),
        in_specs=[pl.BlockSpec(memory_space=pltpu.VMEM)],
        out_specs=pl.BlockSpec(memory_space=pltpu.VMEM),
        scratch_shapes=[
            pltpu.VMEM((2, m, n), jnp.float32),
            pltpu.SemaphoreType.DMA((2,)),
            pltpu.SemaphoreType.DMA((2,)),
        ],
        compiler_params=pltpu.CompilerParams(collective_id=0),
    )(partial)


def kernel(x, Wg, Wu, Wd):
    partial = _compute_partial(x, Wg, Wu, Wd)
    return _ring_allreduce(partial)


# device time: 171769 ns/iter; 4.3115x vs baseline; 4.3115x over previous
import jax
import jax.numpy as jnp
from jax import lax
from jax.experimental import pallas as pl
from jax.experimental.pallas import tpu as pltpu

N_DEV = 16


def _compute_partial(x, Wg, Wu, Wd):
    m, d = x.shape
    h_per = Wg.shape[1]
    n_out = Wd.shape[1]
    nk = 8
    kblk = h_per // nk

    def body(x_ref, wg_ref, wu_ref, wd_ref, out_ref):
        k = pl.program_id(0)
        gate = jnp.dot(x_ref[...], wg_ref[...], preferred_element_type=jnp.float32)
        up = jnp.dot(x_ref[...], wu_ref[...], preferred_element_type=jnp.float32)
        h = gate * (up * jax.nn.sigmoid(up))
        acc = jnp.dot(h, wd_ref[...], preferred_element_type=jnp.float32)

        @pl.when(k == 0)
        def _():
            out_ref[...] = acc

        @pl.when(k != 0)
        def _():
            out_ref[...] += acc

    return pl.pallas_call(
        body,
        grid=(nk,),
        in_specs=[
            pl.BlockSpec((m, d), lambda k: (0, 0)),
            pl.BlockSpec((d, kblk), lambda k: (0, k)),
            pl.BlockSpec((d, kblk), lambda k: (0, k)),
            pl.BlockSpec((kblk, n_out), lambda k: (k, 0)),
        ],
        out_specs=pl.BlockSpec((m, n_out), lambda k: (0, 0)),
        out_shape=jax.ShapeDtypeStruct((m, n_out), jnp.float32),
        compiler_params=pltpu.CompilerParams(
            dimension_semantics=("arbitrary",),
        ),
    )(x, Wg, Wu, Wd)


def _ring_allreduce(partial):
    m, n = partial.shape
    chunk = m // N_DEV
    nstep = N_DEV - 1

    def body(p_ref, out_ref, comm_ref, rs_send, rs_recv, ag_send, ag_recv):
        my = lax.axis_index("i")
        left = lax.rem(my + N_DEV - 1, N_DEV)
        right = lax.rem(my + 1, N_DEV)

        barrier_sem = pltpu.get_barrier_semaphore()
        for nbr in (left, right):
            pl.semaphore_signal(
                barrier_sem,
                inc=1,
                device_id=(nbr,),
                device_id_type=pl.DeviceIdType.MESH,
            )
        pl.semaphore_wait(barrier_sem, 2)

        out_ref[...] = p_ref[...]

        for s in range(nstep):
            send_c = lax.rem(my - s + 2 * N_DEV, N_DEV)
            recv_c = lax.rem(my - s - 1 + 2 * N_DEV, N_DEV)
            rdma = pltpu.make_async_remote_copy(
                src_ref=out_ref.at[pl.ds(send_c * chunk, chunk), :],
                dst_ref=comm_ref.at[s],
                send_sem=rs_send.at[s],
                recv_sem=rs_recv.at[s],
                device_id=(right,),
                device_id_type=pl.DeviceIdType.MESH,
            )
            rdma.start()
            rdma.wait()
            out_ref[pl.ds(recv_c * chunk, chunk), :] += comm_ref[s]

        for s in range(nstep):
            send_c = lax.rem(my + 1 - s + 2 * N_DEV, N_DEV)
            rdma = pltpu.make_async_remote_copy(
                src_ref=out_ref.at[pl.ds(send_c * chunk, chunk), :],
                dst_ref=out_ref.at[pl.ds(send_c * chunk, chunk), :],
                send_sem=ag_send.at[s],
                recv_sem=ag_recv.at[s],
                device_id=(right,),
                device_id_type=pl.DeviceIdType.MESH,
            )
            rdma.start()
            rdma.wait()

    return pl.pallas_call(
        body,
        out_shape=jax.ShapeDtypeStruct((m, n), jnp.float32),
        in_specs=[pl.BlockSpec(memory_space=pltpu.VMEM)],
        out_specs=pl.BlockSpec(memory_space=pltpu.VMEM),
        scratch_shapes=[
            pltpu.VMEM((nstep, chunk, n), jnp.float32),
            pltpu.SemaphoreType.DMA((nstep,)),
            pltpu.SemaphoreType.DMA((nstep,)),
            pltpu.SemaphoreType.DMA((nstep,)),
            pltpu.SemaphoreType.DMA((nstep,)),
        ],
        compiler_params=pltpu.CompilerParams(collective_id=0),
    )(partial)


def kernel(x, Wg, Wu, Wd):
    partial = _compute_partial(x, Wg, Wu, Wd)
    return _ring_allreduce(partial)


# device time: 147354 ns/iter; 5.0259x vs baseline; 1.1657x over previous
import jax
import jax.numpy as jnp
from jax import lax
from jax.experimental import pallas as pl
from jax.experimental.pallas import tpu as pltpu

N_DEV = 16


def _compute_partial(x, Wg, Wu, Wd):
    m, d = x.shape
    h_per = Wg.shape[1]
    n_out = Wd.shape[1]
    nk = 8
    kblk = h_per // nk

    def body(x_ref, wg_ref, wu_ref, wd_ref, out_ref):
        k = pl.program_id(0)
        gate = jnp.dot(x_ref[...], wg_ref[...], preferred_element_type=jnp.float32)
        up = jnp.dot(x_ref[...], wu_ref[...], preferred_element_type=jnp.float32)
        h = gate * (up * jax.nn.sigmoid(up))
        acc = jnp.dot(h, wd_ref[...], preferred_element_type=jnp.float32)

        @pl.when(k == 0)
        def _():
            out_ref[...] = acc

        @pl.when(k != 0)
        def _():
            out_ref[...] += acc

    return pl.pallas_call(
        body,
        grid=(nk,),
        in_specs=[
            pl.BlockSpec((m, d), lambda k: (0, 0)),
            pl.BlockSpec((d, kblk), lambda k: (0, k)),
            pl.BlockSpec((d, kblk), lambda k: (0, k)),
            pl.BlockSpec((kblk, n_out), lambda k: (k, 0)),
        ],
        out_specs=pl.BlockSpec((m, n_out), lambda k: (0, 0)),
        out_shape=jax.ShapeDtypeStruct((m, n_out), jnp.float32),
        compiler_params=pltpu.CompilerParams(
            dimension_semantics=("arbitrary",),
        ),
    )(x, Wg, Wu, Wd)


def _ring_allreduce(partial):
    m, n = partial.shape
    chunk = m // N_DEV
    nstep = N_DEV - 1

    def body(p_ref, out_ref, comm_ref, rs_send, rs_recv, ag_send, ag_recv):
        my = lax.axis_index("i")
        left = lax.rem(my + N_DEV - 1, N_DEV)
        right = lax.rem(my + 1, N_DEV)

        barrier_sem = pltpu.get_barrier_semaphore()
        for nbr in (left, right):
            pl.semaphore_signal(
                barrier_sem,
                inc=1,
                device_id=(nbr,),
                device_id_type=pl.DeviceIdType.MESH,
            )
        pl.semaphore_wait(barrier_sem, 2)

        out_ref[...] = p_ref[...]

        for s in range(nstep):
            send_c = lax.rem(my - s + 2 * N_DEV, N_DEV)
            recv_c = lax.rem(my - s - 1 + 2 * N_DEV, N_DEV)
            rdma = pltpu.make_async_remote_copy(
                src_ref=out_ref.at[pl.ds(send_c * chunk, chunk), :],
                dst_ref=comm_ref.at[s],
                send_sem=rs_send.at[s],
                recv_sem=rs_recv.at[s],
                device_id=(right,),
                device_id_type=pl.DeviceIdType.MESH,
            )
            rdma.start()
            rdma.wait()
            out_ref[pl.ds(recv_c * chunk, chunk), :] += comm_ref[s]

        for s in range(nstep):
            send_c = lax.rem(my + 1 - s + 2 * N_DEV, N_DEV)
            rdma = pltpu.make_async_remote_copy(
                src_ref=out_ref.at[pl.ds(send_c * chunk, chunk), :],
                dst_ref=out_ref.at[pl.ds(send_c * chunk, chunk), :],
                send_sem=ag_send.at[s],
                recv_sem=ag_recv.at[s],
                device_id=(right,),
                device_id_type=pl.DeviceIdType.MESH,
            )
            rdma.start()
            rdma.wait()

    return pl.pallas_call(
        body,
        out_shape=jax.ShapeDtypeStruct((m, n), jnp.float32),
        in_specs=[pl.BlockSpec(memory_space=pltpu.VMEM)],
        out_specs=pl.BlockSpec(memory_space=pltpu.VMEM),
        scratch_shapes=[
            pltpu.VMEM((nstep, chunk, n), jnp.float32),
            pltpu.SemaphoreType.DMA((nstep,)),
            pltpu.SemaphoreType.DMA((nstep,)),
            pltpu.SemaphoreType.DMA((nstep,)),
            pltpu.SemaphoreType.DMA((nstep,)),
        ],
        compiler_params=pltpu.CompilerParams(collective_id=0),
    )(partial)


def kernel(x, Wg, Wu, Wd):
    import os

    mode = os.environ.get("KERNEL_SPLIT", "")
    if mode == "compute":
        return _compute_partial(x, Wg, Wu, Wd)
    if mode == "ar":
        return _ring_allreduce(x)
    partial = _compute_partial(x, Wg, Wu, Wd)
    return _ring_allreduce(partial)


# device time: 129284 ns/iter; 5.7284x vs baseline; 1.1398x over previous
import jax
import jax.numpy as jnp
from jax import lax
from jax.experimental import pallas as pl
from jax.experimental.pallas import tpu as pltpu

N_DEV = 16


def _compute_partial(x, Wg, Wu, Wd):
    m, d = x.shape
    h_per = Wg.shape[1]
    n_out = Wd.shape[1]
    nk = 8
    kblk = h_per // nk

    def body(x_ref, wg_ref, wu_ref, wd_ref, out_ref):
        k = pl.program_id(0)
        gate = jnp.dot(x_ref[...], wg_ref[...], preferred_element_type=jnp.float32)
        up = jnp.dot(x_ref[...], wu_ref[...], preferred_element_type=jnp.float32)
        h = gate * (up * jax.nn.sigmoid(up))
        acc = jnp.dot(h, wd_ref[...], preferred_element_type=jnp.float32)

        @pl.when(k == 0)
        def _():
            out_ref[...] = acc

        @pl.when(k != 0)
        def _():
            out_ref[...] += acc

    return pl.pallas_call(
        body,
        grid=(nk,),
        in_specs=[
            pl.BlockSpec((m, d), lambda k: (0, 0)),
            pl.BlockSpec((d, kblk), lambda k: (0, k)),
            pl.BlockSpec((d, kblk), lambda k: (0, k)),
            pl.BlockSpec((kblk, n_out), lambda k: (k, 0)),
        ],
        out_specs=pl.BlockSpec((m, n_out), lambda k: (0, 0)),
        out_shape=jax.ShapeDtypeStruct((m, n_out), jnp.float32),
        compiler_params=pltpu.CompilerParams(
            dimension_semantics=("arbitrary",),
        ),
    )(x, Wg, Wu, Wd)


def _ring_allreduce(partial):
    m, n = partial.shape
    chunk = m // N_DEV
    half = N_DEV // 2

    def rows(ref, c):
        return ref.at[pl.ds(c * chunk, chunk), :]

    def body(
        p_ref,
        out_ref,
        rcomm,
        lcomm,
        rs_r_send,
        rs_r_recv,
        rs_l_send,
        rs_l_recv,
        ag_r_send,
        ag_r_recv,
        ag_l_send,
        ag_l_recv,
    ):
        my = lax.axis_index("i")
        left = lax.rem(my + N_DEV - 1, N_DEV)
        right = lax.rem(my + 1, N_DEV)

        def cidx(k):
            return lax.rem(my + k + 2 * N_DEV, N_DEV)

        barrier_sem = pltpu.get_barrier_semaphore()
        for nbr in (left, right):
            pl.semaphore_signal(
                barrier_sem,
                inc=1,
                device_id=(nbr,),
                device_id_type=pl.DeviceIdType.MESH,
            )
        pl.semaphore_wait(barrier_sem, 2)

        out_ref[...] = p_ref[...]

        for s in range(half):
            r_rdma = pltpu.make_async_remote_copy(
                src_ref=rows(out_ref, cidx(half - s)),
                dst_ref=rcomm.at[s],
                send_sem=rs_r_send.at[s],
                recv_sem=rs_r_recv.at[s],
                device_id=(right,),
                device_id_type=pl.DeviceIdType.MESH,
            )
            r_rdma.start()
            if s < half - 1:
                l_rdma = pltpu.make_async_remote_copy(
                    src_ref=rows(out_ref, cidx(-(half - 1) + s)),
                    dst_ref=lcomm.at[s],
                    send_sem=rs_l_send.at[s],
                    recv_sem=rs_l_recv.at[s],
                    device_id=(left,),
                    device_id_type=pl.DeviceIdType.MESH,
                )
                l_rdma.start()
            r_rdma.wait()
            rows(out_ref, cidx(half - 1 - s))[...] += rcomm[s]
            if s < half - 1:
                l_rdma.wait()
                rows(out_ref, cidx(-(half - 2) + s))[...] += lcomm[s]

        for s in range(half):
            c_r = cidx(-s)
            r_rdma = pltpu.make_async_remote_copy(
                src_ref=rows(out_ref, c_r),
                dst_ref=rows(out_ref, c_r),
                send_sem=ag_r_send.at[s],
                recv_sem=ag_r_recv.at[s],
                device_id=(right,),
                device_id_type=pl.DeviceIdType.MESH,
            )
            r_rdma.start()
            if s < half - 1:
                c_l = cidx(s)
                l_rdma = pltpu.make_async_remote_copy(
                    src_ref=rows(out_ref, c_l),
                    dst_ref=rows(out_ref, c_l),
                    send_sem=ag_l_send.at[s],
                    recv_sem=ag_l_recv.at[s],
                    device_id=(left,),
                    device_id_type=pl.DeviceIdType.MESH,
                )
                l_rdma.start()
            r_rdma.wait()
            if s < half - 1:
                l_rdma.wait()

    return pl.pallas_call(
        body,
        out_shape=jax.ShapeDtypeStruct((m, n), jnp.float32),
        in_specs=[pl.BlockSpec(memory_space=pltpu.VMEM)],
        out_specs=pl.BlockSpec(memory_space=pltpu.VMEM),
        scratch_shapes=[
            pltpu.VMEM((half, chunk, n), jnp.float32),
            pltpu.VMEM((half - 1, chunk, n), jnp.float32),
            pltpu.SemaphoreType.DMA((half,)),
            pltpu.SemaphoreType.DMA((half,)),
            pltpu.SemaphoreType.DMA((half - 1,)),
            pltpu.SemaphoreType.DMA((half - 1,)),
            pltpu.SemaphoreType.DMA((half,)),
            pltpu.SemaphoreType.DMA((half,)),
            pltpu.SemaphoreType.DMA((half - 1,)),
            pltpu.SemaphoreType.DMA((half - 1,)),
        ],
        compiler_params=pltpu.CompilerParams(collective_id=0),
    )(partial)


def kernel(x, Wg, Wu, Wd):
    import os

    mode = os.environ.get("KERNEL_SPLIT", "")
    if mode == "compute":
        return _compute_partial(x, Wg, Wu, Wd)
    if mode == "ar":
        return _ring_allreduce(x)
    partial = _compute_partial(x, Wg, Wu, Wd)
    return _ring_allreduce(partial)


# device time: 109400 ns/iter; 6.7696x vs baseline; 1.1818x over previous
import jax
import jax.numpy as jnp
from jax import lax
from jax.experimental import pallas as pl
from jax.experimental.pallas import tpu as pltpu

N_DEV = 16


def _compute_partial(x, Wg, Wu, Wd):
    m, d = x.shape
    h_per = Wg.shape[1]
    n_out = Wd.shape[1]
    nk = 8
    kblk = h_per // nk

    def body(x_ref, wg_ref, wu_ref, wd_ref, out_ref):
        k = pl.program_id(0)
        gate = jnp.dot(x_ref[...], wg_ref[...], preferred_element_type=jnp.float32)
        up = jnp.dot(x_ref[...], wu_ref[...], preferred_element_type=jnp.float32)
        h = gate * (up * jax.nn.sigmoid(up))
        acc = jnp.dot(h, wd_ref[...], preferred_element_type=jnp.float32)

        @pl.when(k == 0)
        def _():
            out_ref[...] = acc

        @pl.when(k != 0)
        def _():
            out_ref[...] += acc

    return pl.pallas_call(
        body,
        grid=(nk,),
        in_specs=[
            pl.BlockSpec((m, d), lambda k: (0, 0)),
            pl.BlockSpec((d, kblk), lambda k: (0, k)),
            pl.BlockSpec((d, kblk), lambda k: (0, k)),
            pl.BlockSpec((kblk, n_out), lambda k: (k, 0)),
        ],
        out_specs=pl.BlockSpec((m, n_out), lambda k: (0, 0)),
        out_shape=jax.ShapeDtypeStruct((m, n_out), jnp.float32),
        compiler_params=pltpu.CompilerParams(
            dimension_semantics=("arbitrary",),
        ),
    )(x, Wg, Wu, Wd)


def _ring_allreduce(partial):
    m, n = partial.shape
    chunk = m // N_DEV
    half = N_DEV // 2

    def rows(ref, c):
        return ref.at[pl.ds(c * chunk, chunk), :]

    def body(
        p_ref,
        out_ref,
        rcomm,
        lcomm,
        rs_r_send,
        rs_r_recv,
        rs_l_send,
        rs_l_recv,
        ag_r_send,
        ag_r_recv,
        ag_l_send,
        ag_l_recv,
    ):
        my = lax.axis_index("i")
        left = lax.rem(my + N_DEV - 1, N_DEV)
        right = lax.rem(my + 1, N_DEV)

        def cidx(k):
            return lax.rem(my + k + 2 * N_DEV, N_DEV)

        barrier_sem = pltpu.get_barrier_semaphore()
        for nbr in (left, right):
            pl.semaphore_signal(
                barrier_sem,
                inc=1,
                device_id=(nbr,),
                device_id_type=pl.DeviceIdType.MESH,
            )
        pl.semaphore_wait(barrier_sem, 2)

        out_ref[...] = p_ref[...]

        def rs_r_rdma(s):
            return pltpu.make_async_remote_copy(
                src_ref=rows(out_ref, cidx(half - s)),
                dst_ref=rcomm.at[s],
                send_sem=rs_r_send.at[s],
                recv_sem=rs_r_recv.at[s],
                device_id=(right,),
                device_id_type=pl.DeviceIdType.MESH,
            )

        def rs_l_rdma(s):
            return pltpu.make_async_remote_copy(
                src_ref=rows(out_ref, cidx(-(half - 1) + s)),
                dst_ref=lcomm.at[s],
                send_sem=rs_l_send.at[s],
                recv_sem=rs_l_recv.at[s],
                device_id=(left,),
                device_id_type=pl.DeviceIdType.MESH,
            )

        def ag_r_rdma(s):
            c = cidx(-s)
            return pltpu.make_async_remote_copy(
                src_ref=rows(out_ref, c),
                dst_ref=rows(out_ref, c),
                send_sem=ag_r_send.at[s],
                recv_sem=ag_r_recv.at[s],
                device_id=(right,),
                device_id_type=pl.DeviceIdType.MESH,
            )

        def ag_l_rdma(s):
            c = cidx(s)
            return pltpu.make_async_remote_copy(
                src_ref=rows(out_ref, c),
                dst_ref=rows(out_ref, c),
                send_sem=ag_l_send.at[s],
                recv_sem=ag_l_recv.at[s],
                device_id=(left,),
                device_id_type=pl.DeviceIdType.MESH,
            )

        pending = []

        r0 = rs_r_rdma(0)
        r0.start()
        pending.append(r0)
        l0 = rs_l_rdma(0)
        l0.start()
        pending.append(l0)
        for s in range(half):
            rs_r_rdma(s).wait_recv()
            rows(out_ref, cidx(half - 1 - s))[...] += rcomm[s]
            if s < half - 1:
                nxt = rs_r_rdma(s + 1)
                nxt.start()
                pending.append(nxt)
                rs_l_rdma(s).wait_recv()
                rows(out_ref, cidx(-(half - 2) + s))[...] += lcomm[s]
                if s < half - 2:
                    nxt = rs_l_rdma(s + 1)
                    nxt.start()
                    pending.append(nxt)

        a0 = ag_r_rdma(0)
        a0.start()
        pending.append(a0)
        b0 = ag_l_rdma(0)
        b0.start()
        pending.append(b0)
        for s in range(half):
            ag_r_rdma(s).wait_recv()
            if s < half - 1:
                nxt = ag_r_rdma(s + 1)
                nxt.start()
                pending.append(nxt)
                ag_l_rdma(s).wait_recv()
                if s < half - 2:
                    nxt = ag_l_rdma(s + 1)
                    nxt.start()
                    pending.append(nxt)

        for rdma in pending:
            rdma.wait_send()

    return pl.pallas_call(
        body,
        out_shape=jax.ShapeDtypeStruct((m, n), jnp.float32),
        in_specs=[pl.BlockSpec(memory_space=pltpu.VMEM)],
        out_specs=pl.BlockSpec(memory_space=pltpu.VMEM),
        scratch_shapes=[
            pltpu.VMEM((half, chunk, n), jnp.float32),
            pltpu.VMEM((half - 1, chunk, n), jnp.float32),
            pltpu.SemaphoreType.DMA((half,)),
            pltpu.SemaphoreType.DMA((half,)),
            pltpu.SemaphoreType.DMA((half - 1,)),
            pltpu.SemaphoreType.DMA((half - 1,)),
            pltpu.SemaphoreType.DMA((half,)),
            pltpu.SemaphoreType.DMA((half,)),
            pltpu.SemaphoreType.DMA((half - 1,)),
            pltpu.SemaphoreType.DMA((half - 1,)),
        ],
        compiler_params=pltpu.CompilerParams(collective_id=0),
    )(partial)


def kernel(x, Wg, Wu, Wd):
    import os

    mode = os.environ.get("KERNEL_SPLIT", "")
    if mode == "compute":
        return _compute_partial(x, Wg, Wu, Wd)
    if mode == "ar":
        return _ring_allreduce(x)
    partial = _compute_partial(x, Wg, Wu, Wd)
    return _ring_allreduce(partial)


# device time: 97979 ns/iter; 7.5587x vs baseline; 1.1166x over previous
import jax
import jax.numpy as jnp
from jax import lax
from jax.experimental import pallas as pl
from jax.experimental.pallas import tpu as pltpu

N_DEV = 16


def _compute_partial(x, Wg, Wu, Wd):
    m, d = x.shape
    h_per = Wg.shape[1]
    n_out = Wd.shape[1]
    nk = 8
    kblk = h_per // nk

    def body(x_ref, wg_ref, wu_ref, wd_ref, out_ref):
        k = pl.program_id(0)
        gate = jnp.dot(x_ref[...], wg_ref[...], preferred_element_type=jnp.float32)
        up = jnp.dot(x_ref[...], wu_ref[...], preferred_element_type=jnp.float32)
        h = gate * (up * jax.nn.sigmoid(up))
        acc = jnp.dot(h, wd_ref[...], preferred_element_type=jnp.float32)

        @pl.when(k == 0)
        def _():
            out_ref[...] = acc

        @pl.when(k != 0)
        def _():
            out_ref[...] += acc

    return pl.pallas_call(
        body,
        grid=(nk,),
        in_specs=[
            pl.BlockSpec((m, d), lambda k: (0, 0)),
            pl.BlockSpec((d, kblk), lambda k: (0, k)),
            pl.BlockSpec((d, kblk), lambda k: (0, k)),
            pl.BlockSpec((kblk, n_out), lambda k: (k, 0)),
        ],
        out_specs=pl.BlockSpec((m, n_out), lambda k: (0, 0)),
        out_shape=jax.ShapeDtypeStruct((m, n_out), jnp.float32),
        compiler_params=pltpu.CompilerParams(
            dimension_semantics=("arbitrary",),
        ),
    )(x, Wg, Wu, Wd)


def _ring_allreduce(partial):
    m, n = partial.shape
    chunk = m // N_DEV
    half = N_DEV // 2
    sub = 2
    r_rows = chunk // sub

    def rows(ref, c):
        return ref.at[pl.ds(c * chunk, chunk), :]

    def subrows(ref, c, h):
        return ref.at[pl.ds(c * chunk + h * r_rows, r_rows), :]

    def body(
        p_ref,
        out_ref,
        rcomm,
        lcomm,
        rs_r_send,
        rs_r_recv,
        rs_l_send,
        rs_l_recv,
        ag_r_send,
        ag_r_recv,
        ag_l_send,
        ag_l_recv,
    ):
        my = lax.axis_index("i")
        left = lax.rem(my + N_DEV - 1, N_DEV)
        right = lax.rem(my + 1, N_DEV)

        def cidx(k):
            return lax.rem(my + k + 2 * N_DEV, N_DEV)

        barrier_sem = pltpu.get_barrier_semaphore()
        for nbr in (left, right):
            pl.semaphore_signal(
                barrier_sem,
                inc=1,
                device_id=(nbr,),
                device_id_type=pl.DeviceIdType.MESH,
            )
        pl.semaphore_wait(barrier_sem, 2)

        out_ref[...] = p_ref[...]

        def rs_r_rdma(s, h):
            return pltpu.make_async_remote_copy(
                src_ref=subrows(out_ref, cidx(half - s), h),
                dst_ref=rcomm.at[s * sub + h],
                send_sem=rs_r_send.at[s * sub + h],
                recv_sem=rs_r_recv.at[s * sub + h],
                device_id=(right,),
                device_id_type=pl.DeviceIdType.MESH,
            )

        def rs_l_rdma(s, h):
            return pltpu.make_async_remote_copy(
                src_ref=subrows(out_ref, cidx(-(half - 1) + s), h),
                dst_ref=lcomm.at[s * sub + h],
                send_sem=rs_l_send.at[s * sub + h],
                recv_sem=rs_l_recv.at[s * sub + h],
                device_id=(left,),
                device_id_type=pl.DeviceIdType.MESH,
            )

        def ag_r_rdma(s, h):
            c = cidx(-s)
            return pltpu.make_async_remote_copy(
                src_ref=subrows(out_ref, c, h),
                dst_ref=subrows(out_ref, c, h),
                send_sem=ag_r_send.at[s * sub + h],
                recv_sem=ag_r_recv.at[s * sub + h],
                device_id=(right,),
                device_id_type=pl.DeviceIdType.MESH,
            )

        def ag_l_rdma(s, h):
            c = cidx(s)
            return pltpu.make_async_remote_copy(
                src_ref=subrows(out_ref, c, h),
                dst_ref=subrows(out_ref, c, h),
                send_sem=ag_l_send.at[s * sub + h],
                recv_sem=ag_l_recv.at[s * sub + h],
                device_id=(left,),
                device_id_type=pl.DeviceIdType.MESH,
            )

        pending = []

        def start(rdma):
            rdma.start()
            pending.append(rdma)

        for h in range(sub):
            start(rs_r_rdma(0, h))
            start(rs_l_rdma(0, h))
        for s in range(half):
            for h in range(sub):
                rs_r_rdma(s, h).wait_recv()
                subrows(out_ref, cidx(half - 1 - s), h)[...] += rcomm[s * sub + h]
                if s < half - 1:
                    start(rs_r_rdma(s + 1, h))
            if s < half - 1:
                for h in range(sub):
                    rs_l_rdma(s, h).wait_recv()
                    subrows(out_ref, cidx(-(half - 2) + s), h)[...] += lcomm[
                        s * sub + h
                    ]
                    if s < half - 2:
                        start(rs_l_rdma(s + 1, h))

        for h in range(sub):
            start(ag_r_rdma(0, h))
            start(ag_l_rdma(0, h))
        for s in range(half):
            for h in range(sub):
                ag_r_rdma(s, h).wait_recv()
                if s < half - 1:
                    start(ag_r_rdma(s + 1, h))
            if s < half - 1:
                for h in range(sub):
                    ag_l_rdma(s, h).wait_recv()
                    if s < half - 2:
                        start(ag_l_rdma(s + 1, h))

        for rdma in pending:
            rdma.wait_send()

    return pl.pallas_call(
        body,
        out_shape=jax.ShapeDtypeStruct((m, n), jnp.float32),
        in_specs=[pl.BlockSpec(memory_space=pltpu.VMEM)],
        out_specs=pl.BlockSpec(memory_space=pltpu.VMEM),
        scratch_shapes=[
            pltpu.VMEM((half * sub, r_rows, n), jnp.float32),
            pltpu.VMEM(((half - 1) * sub, r_rows, n), jnp.float32),
            pltpu.SemaphoreType.DMA((half * sub,)),
            pltpu.SemaphoreType.DMA((half * sub,)),
            pltpu.SemaphoreType.DMA(((half - 1) * sub,)),
            pltpu.SemaphoreType.DMA(((half - 1) * sub,)),
            pltpu.SemaphoreType.DMA((half * sub,)),
            pltpu.SemaphoreType.DMA((half * sub,)),
            pltpu.SemaphoreType.DMA(((half - 1) * sub,)),
            pltpu.SemaphoreType.DMA(((half - 1) * sub,)),
        ],
        compiler_params=pltpu.CompilerParams(collective_id=0),
    )(partial)


def kernel(x, Wg, Wu, Wd):
    import os

    mode = os.environ.get("KERNEL_SPLIT", "")
    if mode == "compute":
        return _compute_partial(x, Wg, Wu, Wd)
    if mode == "ar":
        return _ring_allreduce(x)
    partial = _compute_partial(x, Wg, Wu, Wd)
    return _ring_allreduce(partial)
